# Initial kernel scaffold; baseline (speedup 1.0000x reference)
import jax
import jax.numpy as jnp
from jax.experimental import pallas as pl


def kernel(node_type_ids, edge_index, ntype_embed):
    raise NotImplementedError("write your pallas kernel here")


# speedup vs baseline: 31.3582x; 31.3582x over previous
import dataclasses
import functools

import jax
import jax.numpy as jnp
from jax import lax
from jax.experimental import pallas as pl
from jax.experimental.pallas import tpu as pltpu
from jax.experimental.pallas import tpu_sc as plsc

N_NODES = 100000
N_EDGES = 3200000
N_TYPES = 8
EMBED_DIM = 16

NC = 2
NS = 16
NW = NC * NS
L = 16

CB = 128
NCHUNKS = N_EDGES // CB
CH_PER_TILE = -(-NCHUNKS // NW)

KS = N_NODES * N_TYPES


def _mesh():
    return plsc.VectorSubcoreMesh(
        core_axis_name="c", subcore_axis_name="s", num_cores=NC, num_subcores=NS
    )


def _sc_params(tc_tiling=True):
    cp = pltpu.CompilerParams()
    if "needs_layout_passes" in pltpu.CompilerParams.__dataclass_fields__:
        cp = dataclasses.replace(cp, needs_layout_passes=False)
    if not tc_tiling:
        cp = dataclasses.replace(cp, use_tc_tiling_on_sc=False)
    return cp


ZB = 10000


def _hist_body(packed_hbm, edges_hbm, zeros_hbm, out_hbm,
               packed_v, src_v, dst_v, k1_v, k2_v, ones_v, zb_v, acc_sh):
    cid = lax.axis_index("c")
    sid = lax.axis_index("s")
    wid = sid * NC + cid

    pltpu.sync_copy(packed_hbm, packed_v)
    @pl.loop(0, CB, step=L)
    def _(j):
        ones_v[0, pl.ds(j, L)] = jnp.ones((L,), jnp.float32)

    zn = KS // NS
    pltpu.sync_copy(zeros_hbm.at[pl.ds(0, ZB)], zb_v)

    @pl.loop(0, zn, step=ZB)
    def _(k):
        pltpu.sync_copy(zb_v, acc_sh.at[pl.ds(sid * zn + k, ZB)])

    plsc.subcore_barrier()

    @pl.loop(0, CH_PER_TILE)
    def _(i):
        ch = wid + i * NW

        @pl.when(ch < NCHUNKS)
        def _():
            off = ch * CB
            pltpu.sync_copy(edges_hbm.at[pl.ds(0, 1), pl.ds(off, CB)], src_v)
            pltpu.sync_copy(edges_hbm.at[pl.ds(1, 1), pl.ds(off, CB)], dst_v)

            @pl.loop(0, CB, step=L)
            def _(j):
                s = src_v[0, pl.ds(j, L)]
                d = dst_v[0, pl.ds(j, L)]
                ws = plsc.load_gather(packed_v, [lax.shift_right_logical(s, 2)])
                wd = plsc.load_gather(packed_v, [lax.shift_right_logical(d, 2)])
                ts = lax.shift_right_logical(
                    ws, lax.shift_left(s & 3, 3)) & 0xFF
                td = lax.shift_right_logical(
                    wd, lax.shift_left(d & 3, 3)) & 0xFF
                k1_v[0, pl.ds(j, L)] = d * N_TYPES + ts
                k2_v[0, pl.ds(j, L)] = s * N_TYPES + td

            pltpu.sync_copy(ones_v.at[0], acc_sh.at[k1_v.at[0]], add=True)
            pltpu.sync_copy(ones_v.at[0], acc_sh.at[k2_v.at[0]], add=True)

    plsc.subcore_barrier()

    @pl.loop(0, zn, step=ZB)
    def _(k):
        pltpu.sync_copy(acc_sh.at[pl.ds(sid * zn + k, ZB)], zb_v)
        pltpu.sync_copy(zb_v, out_hbm.at[pl.ds(cid * KS + sid * zn + k, ZB)])


def _stage1(packed, edges, zeros_flat):
    kern = pl.kernel(
        _hist_body,
        out_type=jax.ShapeDtypeStruct((NC * KS,), jnp.float32),
        mesh=_mesh(),
        compiler_params=_sc_params(),
        scratch_types=[
            pltpu.VMEM((N_NODES // 4,), jnp.int32),
            pltpu.VMEM((1, CB), jnp.int32),
            pltpu.VMEM((1, CB), jnp.int32),
            pltpu.VMEM((1, CB), jnp.int32),
            pltpu.VMEM((1, CB), jnp.int32),
            pltpu.VMEM((1, CB), jnp.float32),
            pltpu.VMEM((ZB,), jnp.float32),
            pltpu.VMEM_SHARED((KS,), jnp.float32),
        ],
    )
    return kern(packed, edges, zeros_flat)


def _add_body(p_ref, o_ref):
    o_ref[...] = p_ref[0] + p_ref[1]


def _combine(p):
    return pl.pallas_call(
        _add_body,
        out_shape=jax.ShapeDtypeStruct((KS // 128, 128), jnp.float32),
    )(p.reshape(NC, KS // 128, 128))


RB = 1000
NRCH = N_NODES // RB
NRCH_PER_TILE = -(-NRCH // NS)


def _prop_body(c1_hbm, edges_hbm, zeros_hbm, out_hbm,
               src_v, dst_v, rows_a, rows_b, rb_v, acc_sh, sem_a, sem_b):
    cid = lax.axis_index("c")
    sid = lax.axis_index("s")
    wid = sid * NC + cid

    pltpu.sync_copy(zeros_hbm.at[pl.ds(0, RB), :], rb_v)

    @pl.loop(0, NRCH_PER_TILE)
    def _(j):
        rch = sid + j * NS

        @pl.when(rch < NRCH)
        def _():
            pltpu.sync_copy(rb_v, acc_sh.at[pl.ds(rch * RB, RB), :])

    plsc.subcore_barrier()

    @pl.loop(0, CH_PER_TILE)
    def _(i):
        ch = wid + i * NW

        @pl.when(ch < NCHUNKS)
        def _():
            off = ch * CB
            pltpu.sync_copy(edges_hbm.at[pl.ds(0, 1), pl.ds(off, CB)], src_v)
            pltpu.sync_copy(edges_hbm.at[pl.ds(1, 1), pl.ds(off, CB)], dst_v)
            cp_a = pltpu.async_copy(c1_hbm.at[src_v.at[0]], rows_a, sem_a)
            cp_b = pltpu.async_copy(c1_hbm.at[dst_v.at[0]], rows_b, sem_b)
            cp_a.wait()
            cp_b.wait()
            pltpu.sync_copy(rows_a, acc_sh.at[dst_v.at[0]], add=True)
            pltpu.sync_copy(rows_b, acc_sh.at[src_v.at[0]], add=True)

    plsc.subcore_barrier()

    @pl.loop(0, NRCH_PER_TILE)
    def _(j):
        rch = sid + j * NS

        @pl.when(rch < NRCH)
        def _():
            pltpu.sync_copy(acc_sh.at[pl.ds(rch * RB, RB), :], rb_v)
            pltpu.sync_copy(
                rb_v, out_hbm.at[pl.ds(cid * N_NODES + rch * RB, RB), :])


def _stage3(c1, edges, zeros_rows):
    kern = pl.kernel(
        _prop_body,
        out_type=jax.ShapeDtypeStruct((NC * N_NODES, N_TYPES), jnp.float32),
        mesh=_mesh(),
        compiler_params=_sc_params(tc_tiling=False),
        scratch_types=[
            pltpu.VMEM((1, CB), jnp.int32),
            pltpu.VMEM((1, CB), jnp.int32),
            pltpu.VMEM((CB, N_TYPES), jnp.float32),
            pltpu.VMEM((CB, N_TYPES), jnp.float32),
            pltpu.VMEM((RB, N_TYPES), jnp.float32),
            pltpu.VMEM_SHARED((N_NODES, N_TYPES), jnp.float32),
            pltpu.SemaphoreType.DMA,
            pltpu.SemaphoreType.DMA,
        ],
    )
    return kern(c1, edges, zeros_rows)


NB = 1000
NBCHUNKS = N_NODES // NB
NB_PER_TILE = -(-NBCHUNKS // NW)


def _final_body(p2_hbm, types_hbm, emb_hbm, out_hbm,
                a_v, b_v, s_v, t_v, e_v, o_v):
    cid = lax.axis_index("c")
    sid = lax.axis_index("s")
    wid = sid * NC + cid

    pltpu.sync_copy(emb_hbm, e_v)
    e_rows = [e_v[pl.ds(t * EMBED_DIM, L)] for t in range(N_TYPES)]
    lane = lax.iota(jnp.int32, L)

    @pl.loop(0, NB_PER_TILE)
    def _(i):
        ch = wid + i * NW

        @pl.when(ch < NBCHUNKS)
        def _():
            noff = ch * NB
            pltpu.sync_copy(
                p2_hbm.at[pl.ds(noff * N_TYPES, NB * N_TYPES)], a_v)
            pltpu.sync_copy(
                p2_hbm.at[pl.ds(KS + noff * N_TYPES, NB * N_TYPES)], b_v)
            pltpu.sync_copy(types_hbm.at[pl.ds(noff, NB)], t_v)

            @pl.loop(0, NB * N_TYPES, step=L)
            def _(j):
                n_vec = lax.shift_right_logical(j + lane, 3)
                t_rep = plsc.load_gather(t_v, [n_vec])
                f = jnp.where(t_rep == 0, 1.0, 0.0).astype(jnp.float32)
                s_v[pl.ds(j, L)] = (a_v[pl.ds(j, L)] + b_v[pl.ds(j, L)]) * f

            @pl.loop(0, NB)
            def _(n):
                acc = jnp.zeros((L,), jnp.float32)
                for t in range(N_TYPES):
                    st = plsc.load_gather(
                        s_v, [jnp.full((L,), n * N_TYPES + t, jnp.int32)])
                    acc = acc + st * e_rows[t]
                o_v[pl.ds(n * EMBED_DIM, L)] = acc

            pltpu.sync_copy(
                o_v, out_hbm.at[pl.ds(noff * EMBED_DIM, NB * EMBED_DIM)])


def _stage4(p2_flat, types, emb_flat):
    kern = pl.kernel(
        _final_body,
        out_type=jax.ShapeDtypeStruct((N_NODES * EMBED_DIM,), jnp.float32),
        mesh=_mesh(),
        compiler_params=_sc_params(),
        scratch_types=[
            pltpu.VMEM((NB * N_TYPES,), jnp.float32),
            pltpu.VMEM((NB * N_TYPES,), jnp.float32),
            pltpu.VMEM((NB * N_TYPES,), jnp.float32),
            pltpu.VMEM((NB,), jnp.int32),
            pltpu.VMEM((N_TYPES * EMBED_DIM,), jnp.float32),
            pltpu.VMEM((NB * EMBED_DIM,), jnp.float32),
        ],
    )
    return kern(p2_flat, types, emb_flat)


def kernel(node_type_ids, edge_index, ntype_embed):
    types = node_type_ids.astype(jnp.int32)
    edges = edge_index.astype(jnp.int32)
    zeros_flat = jnp.zeros((KS,), jnp.float32)
    t4 = types.reshape(N_NODES // 4, 4)
    packed = t4[:, 0] | (t4[:, 1] << 8) | (t4[:, 2] << 16) | (t4[:, 3] << 24)

    p1 = _stage1(packed, edges, zeros_flat)
    c1 = _combine(p1).reshape(N_NODES, N_TYPES)
    p2 = _stage3(c1, edges, zeros_flat.reshape(N_NODES, N_TYPES))
    out_flat = _stage4(p2.reshape(NC * KS), types, ntype_embed.reshape(-1))
    return out_flat.reshape(N_NODES, EMBED_DIM)
